# baseline (speedup 1.0000x reference)
import functools

import jax
import jax.numpy as jnp
from jax import lax
from jax.experimental import pallas as pl
from jax.experimental.pallas import tpu as pltpu
from jax.experimental.pallas import tpu_sc as plsc

_NC = 1
_NS = 16


@functools.lru_cache(maxsize=None)
def _make_sc_broadcast(B, V):
    rows_sc = B // _NC
    R = 512
    rows_fill = R // _NS
    assert rows_sc % (R * _NS) == 0
    n_out = rows_sc // (_NS * R)

    mesh = plsc.VectorSubcoreMesh(
        core_axis_name="c", subcore_axis_name="s", num_cores=1)

    @functools.partial(
        pl.kernel,
        out_type=jax.ShapeDtypeStruct((B, V), jnp.float32),
        mesh=mesh,
        scratch_types=[
            pltpu.VMEM_SHARED((R, V), jnp.float32),
            pltpu.SemaphoreType.DMA,
        ],
    )
    def broadcast_kernel(table_hbm, out_hbm, shared_buf, sem):
        sid = lax.axis_index("s")
        fills = [
            pltpu.async_copy(table_hbm, shared_buf.at[sid * rows_fill + r], sem)
            for r in range(rows_fill)
        ]
        for cp in fills:
            cp.wait()
        plsc.subcore_barrier()
        base = sid * R
        copies = [
            pltpu.async_copy(
                shared_buf,
                out_hbm.at[pl.ds(base + c * _NS * R, R)],
                sem,
            )
            for c in range(n_out)
        ]
        for cp in copies:
            cp.wait()

    return broadcast_kernel


def kernel(x, table):
    B = x.shape[0]
    V = table.shape[0]
    fn = _make_sc_broadcast(B, V)
    return fn(table.reshape(V))
